# baseline (speedup 1.0000x reference)
import functools

import jax
import jax.numpy as jnp
from jax import lax
from jax.experimental import pallas as pl
from jax.experimental.pallas import tpu as pltpu
from jax.experimental.pallas import tpu_sc as plsc

N = 10000
E = 320000
D_IN = 128
H = 256
HH = 128
L = 4
G = 64
T = 100

NS = 16
EPT = E // NS
CH = 40
NCHUNK = EPT // CH
IB = 10
NBLK = NCHUNK // IB
NPAD = 10240
RPS = NPAD // NS
ZR = 16
DR = 128


def _sc_layer(hlo, hhi, src4, dst4, typ4, elo, ehi):
    mesh = plsc.VectorSubcoreMesh(core_axis_name="c", subcore_axis_name="s")

    @functools.partial(
        pl.kernel,
        out_type=jax.ShapeDtypeStruct((2 * N, HH), jnp.float32),
        mesh=mesh,
        scratch_types=[
            pltpu.VMEM((2, IB, CH), jnp.int32),
            pltpu.VMEM((2, IB, CH), jnp.int32),
            pltpu.VMEM((2, IB, CH), jnp.int32),
            pltpu.VMEM((2, CH, HH), jnp.float32),
            pltpu.VMEM((2, CH, HH), jnp.float32),
            pltpu.VMEM((2, CH, HH), jnp.float32),
            pltpu.VMEM_SHARED((NPAD, HH), jnp.float32),
            pltpu.VMEM_SHARED((104, HH), jnp.float32),
            pltpu.SemaphoreType.DMA, pltpu.SemaphoreType.DMA,
            pltpu.SemaphoreType.DMA, pltpu.SemaphoreType.DMA,
            pltpu.SemaphoreType.DMA, pltpu.SemaphoreType.DMA,
            pltpu.SemaphoreType.DMA,
        ],
    )
    def k(hlo_hbm, hhi_hbm, src_hbm, dst_hbm, typ_hbm, elo_hbm, ehi_hbm,
          out_hbm, sblk, dblk, tblk, hbuf, ebuf, msg, acc, esp,
          gs0, gs1, es0, es1, ss0, ss1, isem):
        gsems = (gs0, gs1)
        esems = (es0, es1)
        ssems = (ss0, ss1)
        c = lax.axis_index("c")
        s = lax.axis_index("s")

        @pl.when(s == 0)
        def _():
            @pl.when(c == 0)
            def _():
                pltpu.sync_copy(elo_hbm, esp.at[pl.ds(0, T)])

            @pl.when(c == 1)
            def _():
                pltpu.sync_copy(ehi_hbm, esp.at[pl.ds(0, T)])

        zero16 = jnp.zeros((16,), jnp.float32)

        @pl.loop(0, CH)
        def _(r):
            for j in range(HH // 16):
                msg[0, r, pl.ds(j * 16, 16)] = zero16

        for kb in range(RPS // CH):
            pltpu.sync_copy(msg.at[0], acc.at[pl.ds(s * RPS + kb * CH, CH)])
        plsc.subcore_barrier()

        def prefetch_block(kb, hh):
            pltpu.async_copy(src_hbm.at[s, kb], sblk.at[hh], isem)
            pltpu.async_copy(dst_hbm.at[s, kb], dblk.at[hh], isem)
            pltpu.async_copy(typ_hbm.at[s, kb], tblk.at[hh], isem)

        def wait_block():
            pltpu.make_async_copy(src_hbm.at[s, 0], sblk.at[0], isem).wait()
            pltpu.make_async_copy(dst_hbm.at[s, 0], dblk.at[0], isem).wait()
            pltpu.make_async_copy(typ_hbm.at[s, 0], tblk.at[0], isem).wait()

        def issue_g(hh, rj, b):
            pltpu.async_copy(esp.at[tblk.at[hh, rj]], ebuf.at[b], esems[b])

            @pl.when(c == 0)
            def _():
                pltpu.async_copy(hlo_hbm.at[sblk.at[hh, rj]], hbuf.at[b],
                                 gsems[b])

            @pl.when(c == 1)
            def _():
                pltpu.async_copy(hhi_hbm.at[sblk.at[hh, rj]], hbuf.at[b],
                                 gsems[b])

        def wait_g(b):
            pltpu.make_async_copy(hlo_hbm.at[sblk.at[0, 0]], hbuf.at[b],
                                  gsems[b]).wait()
            pltpu.make_async_copy(esp.at[tblk.at[0, 0]], ebuf.at[b],
                                  esems[b]).wait()

        def compute(b):
            @plsc.parallel_loop(0, CH, unroll=4)
            def _(r):
                for j in range(HH // 16):
                    sl = pl.ds(j * 16, 16)
                    msg[b, r, sl] = jnp.maximum(
                        hbuf[b, r, sl] + ebuf[b, r, sl], 0.0)

        def issue_s(hh, rj, b):
            pltpu.async_copy(msg.at[b], acc.at[dblk.at[hh, rj]], ssems[b],
                             add=True)

        def wait_s(b):
            pltpu.make_async_copy(msg.at[b], acc.at[dblk.at[0, 0]],
                                  ssems[b]).wait()

        prefetch_block(0, 0)
        wait_block()
        issue_g(0, 0, 0)
        issue_g(0, 1, 1)

        @pl.loop(0, NBLK // 2)
        def _(kb2):
            for pb in range(2):
                kb = 2 * kb2 + pb
                for rj in range(IB):
                    b = rj % 2
                    i = kb * IB + rj
                    wait_g(b)

                    @pl.when(i >= 2)
                    def _():
                        wait_s(b)

                    compute(b)

                    if rj == 1:
                        @pl.when(kb + 1 < NBLK)
                        def _():
                            prefetch_block(kb + 1, (pb + 1) % 2)
                    if rj == IB - 2:
                        @pl.when(kb + 1 < NBLK)
                        def _():
                            wait_block()

                    if rj < IB - 2:
                        g_hh, g_rj = pb, rj + 2
                    else:
                        g_hh, g_rj = (pb + 1) % 2, rj + 2 - IB

                    @pl.when(i + 2 < NCHUNK)
                    def _():
                        issue_g(g_hh, g_rj, b)

                    issue_s(pb, rj, b)

        wait_s(0)
        wait_s(1)
        plsc.subcore_barrier()

        @pl.when(s < NS - 1)
        def _():
            for kblk in range(RPS // DR):
                r0 = s * RPS + kblk * DR
                pltpu.sync_copy(acc.at[pl.ds(r0, DR)],
                                out_hbm.at[pl.ds(c * N + r0, DR)])

        @pl.when(s == NS - 1)
        def _():
            base = (NS - 1) * RPS
            for r0, sz in ((0, 128), (128, 128), (256, 128), (384, 16)):
                pltpu.sync_copy(acc.at[pl.ds(base + r0, sz)],
                                out_hbm.at[pl.ds(c * N + base + r0, sz)])

    return k(hlo, hhi, src4, dst4, typ4, elo, ehi)


def _tc_input(node, W_in, b_in):
    B = 1000

    def body(x_ref, w_ref, b_ref, out_ref):
        z = jnp.dot(x_ref[...], w_ref[...],
                    preferred_element_type=jnp.float32) + b_ref[...]
        out_ref[0] = z[:, :HH]
        out_ref[1] = z[:, HH:]

    return pl.pallas_call(
        body,
        grid=(N // B,),
        in_specs=[
            pl.BlockSpec((B, D_IN), lambda j: (j, 0)),
            pl.BlockSpec((D_IN, H), lambda j: (0, 0)),
            pl.BlockSpec((1, H), lambda j: (0, 0)),
        ],
        out_specs=pl.BlockSpec((2, B, HH), lambda j: (0, j, 0)),
        out_shape=jax.ShapeDtypeStruct((2, N, HH), jnp.float32),
    )(node, W_in, b_in.reshape(1, H))


def _tc_layer(h2, agg2, Wi2, bi):
    B = 1000

    def body(h_ref, a_ref, w_ref, b_ref, out_ref):
        h_lo = h_ref[0]
        h_hi = h_ref[1]
        x_lo = h_lo + a_ref[0]
        x_hi = h_hi + a_ref[1]
        z = (jnp.dot(x_lo, w_ref[0], preferred_element_type=jnp.float32)
             + jnp.dot(x_hi, w_ref[1], preferred_element_type=jnp.float32)
             + b_ref[...])
        z = jnp.where(z > 0, z, 0.01 * z)
        out_ref[0] = z[:, :HH] + h_lo
        out_ref[1] = z[:, HH:] + h_hi

    return pl.pallas_call(
        body,
        grid=(N // B,),
        in_specs=[
            pl.BlockSpec((2, B, HH), lambda j: (0, j, 0)),
            pl.BlockSpec((2, B, HH), lambda j: (0, j, 0)),
            pl.BlockSpec((2, HH, H), lambda j: (0, 0, 0)),
            pl.BlockSpec((1, H), lambda j: (0, 0)),
        ],
        out_specs=pl.BlockSpec((2, B, HH), lambda j: (0, j, 0)),
        out_shape=jax.ShapeDtypeStruct((2, N, HH), jnp.float32),
    )(h2, agg2, Wi2, bi)


def _tc_pool(h2, bi_row, W_out, b_out):
    B = 1000

    def body(h_ref, bi_ref, w_ref, b_ref, out_ref, sums, counts):
        j = pl.program_id(0)

        @pl.when(j == 0)
        def _():
            sums[...] = jnp.zeros_like(sums)
            counts[...] = jnp.zeros_like(counts)

        x = jnp.concatenate([h_ref[0], h_ref[1]], axis=1)
        gids = lax.broadcasted_iota(jnp.int32, (G, B), 0)
        m = (gids == bi_ref[0]).astype(jnp.float32)
        sums[...] += jnp.dot(m, x, preferred_element_type=jnp.float32)
        counts[...] += jnp.sum(m, axis=1, keepdims=True)

        @pl.when(j == pl.num_programs(0) - 1)
        def _():
            mean = sums[...] / jnp.maximum(counts[...], 1.0)
            out_ref[...] = jnp.dot(mean, w_ref[...],
                                   preferred_element_type=jnp.float32) + b_ref[...]

    return pl.pallas_call(
        body,
        grid=(N // B,),
        in_specs=[
            pl.BlockSpec((2, B, HH), lambda j: (0, j, 0)),
            pl.BlockSpec((1, 1, B), lambda j: (j, 0, 0)),
            pl.BlockSpec((H, 1), lambda j: (0, 0)),
            pl.BlockSpec((1, 1), lambda j: (0, 0)),
        ],
        out_specs=pl.BlockSpec((G, 1), lambda j: (0, 0)),
        out_shape=jax.ShapeDtypeStruct((G, 1), jnp.float32),
        scratch_shapes=[
            pltpu.VMEM((G, H), jnp.float32),
            pltpu.VMEM((G, 1), jnp.float32),
        ],
    )(h2, bi_row.reshape(N // B, 1, B), W_out, b_out.reshape(1, 1))


def kernel(node, edge, edge_type, batch_index, W_in, b_in, emb, Wc, bc, W_out, b_out):
    src4 = edge[:, 0].reshape(NS, NBLK, IB, CH)
    dst4 = edge[:, 1].reshape(NS, NBLK, IB, CH)
    typ4 = edge_type[:, 0].reshape(NS, NBLK, IB, CH)
    elo = emb[:, :HH]
    ehi = emb[:, HH:]

    h2 = _tc_input(node, W_in, b_in)
    for i in range(L):
        agg = _sc_layer(h2[0], h2[1], src4, dst4, typ4, elo, ehi)
        h2 = _tc_layer(h2, agg.reshape(2, N, HH),
                       Wc[i].reshape(2, HH, H), bc[i].reshape(1, H))
    out = _tc_pool(h2, batch_index.reshape(1, N), W_out, b_out.reshape(1, 1))
    return out.reshape(G)


# speedup vs baseline: 1.0149x; 1.0149x over previous
import functools

import jax
import jax.numpy as jnp
from jax import lax
from jax.experimental import pallas as pl
from jax.experimental.pallas import tpu as pltpu
from jax.experimental.pallas import tpu_sc as plsc

N = 10000
E = 320000
D_IN = 128
H = 256
HH = 128
L = 4
G = 64
T = 100

NS = 16
EPT = E // NS
CH = 40
NCHUNK = EPT // CH
IB = 10
NBLK = NCHUNK // IB
NPAD = 10240
RPS = NPAD // NS
ZR = 16
DR = 128


def _sc_layer(hlo, hhi, src4, dst4, typ4, elo, ehi):
    mesh = plsc.VectorSubcoreMesh(core_axis_name="c", subcore_axis_name="s")

    @functools.partial(
        pl.kernel,
        out_type=jax.ShapeDtypeStruct((2 * N, HH), jnp.float32),
        mesh=mesh,
        scratch_types=[
            pltpu.VMEM((2, IB, CH), jnp.int32),
            pltpu.VMEM((2, IB, CH), jnp.int32),
            pltpu.VMEM((2, IB, CH), jnp.int32),
            pltpu.VMEM((3, CH, HH), jnp.float32),
            pltpu.VMEM((3, CH, HH), jnp.float32),
            pltpu.VMEM_SHARED((NPAD, HH), jnp.float32),
            pltpu.VMEM_SHARED((104, HH), jnp.float32),
            pltpu.SemaphoreType.DMA, pltpu.SemaphoreType.DMA,
            pltpu.SemaphoreType.DMA, pltpu.SemaphoreType.DMA,
            pltpu.SemaphoreType.DMA, pltpu.SemaphoreType.DMA,
            pltpu.SemaphoreType.DMA, pltpu.SemaphoreType.DMA,
            pltpu.SemaphoreType.DMA, pltpu.SemaphoreType.DMA,
        ],
    )
    def k(hlo_hbm, hhi_hbm, src_hbm, dst_hbm, typ_hbm, elo_hbm, ehi_hbm,
          out_hbm, sblk, dblk, tblk, hbuf, ebuf, acc, esp,
          gs0, gs1, gs2, es0, es1, es2, ss0, ss1, ss2, isem):
        gsems = (gs0, gs1, gs2)
        esems = (es0, es1, es2)
        ssems = (ss0, ss1, ss2)
        c = lax.axis_index("c")
        s = lax.axis_index("s")

        @pl.when(s == 0)
        def _():
            @pl.when(c == 0)
            def _():
                pltpu.sync_copy(elo_hbm, esp.at[pl.ds(0, T)])

            @pl.when(c == 1)
            def _():
                pltpu.sync_copy(ehi_hbm, esp.at[pl.ds(0, T)])

        zero16 = jnp.zeros((16,), jnp.float32)

        @pl.loop(0, CH)
        def _(r):
            for j in range(HH // 16):
                ebuf[0, r, pl.ds(j * 16, 16)] = zero16

        for kb in range(RPS // CH):
            pltpu.sync_copy(ebuf.at[0], acc.at[pl.ds(s * RPS + kb * CH, CH)])
        plsc.subcore_barrier()

        def load_block(kb):
            hh = lax.rem(kb, 2)
            pltpu.sync_copy(src_hbm.at[s, kb], sblk.at[hh])
            pltpu.sync_copy(dst_hbm.at[s, kb], dblk.at[hh])
            pltpu.sync_copy(typ_hbm.at[s, kb], tblk.at[hh])

        def prefetch_block(kb):
            hh = lax.rem(kb, 2)
            pltpu.async_copy(src_hbm.at[s, kb], sblk.at[hh], isem)
            pltpu.async_copy(dst_hbm.at[s, kb], dblk.at[hh], isem)
            pltpu.async_copy(typ_hbm.at[s, kb], tblk.at[hh], isem)

        def wait_block():
            pltpu.make_async_copy(src_hbm.at[s, 0], sblk.at[0], isem).wait()
            pltpu.make_async_copy(dst_hbm.at[s, 0], dblk.at[0], isem).wait()
            pltpu.make_async_copy(typ_hbm.at[s, 0], tblk.at[0], isem).wait()

        def issue_g(j, b):
            hj = lax.rem(lax.div(j, IB), 2)
            rj = lax.rem(j, IB)
            pltpu.async_copy(esp.at[tblk.at[hj, rj]], ebuf.at[b], esems[b])

            @pl.when(c == 0)
            def _():
                pltpu.async_copy(hlo_hbm.at[sblk.at[hj, rj]], hbuf.at[b],
                                 gsems[b])

            @pl.when(c == 1)
            def _():
                pltpu.async_copy(hhi_hbm.at[sblk.at[hj, rj]], hbuf.at[b],
                                 gsems[b])

        def wait_g(b):
            pltpu.make_async_copy(hlo_hbm.at[sblk.at[0, 0]], hbuf.at[b],
                                  gsems[b]).wait()
            pltpu.make_async_copy(esp.at[tblk.at[0, 0]], ebuf.at[b],
                                  esems[b]).wait()

        def compute(b):
            @plsc.parallel_loop(0, CH, unroll=8)
            def _(r):
                for j in range(HH // 16):
                    sl = pl.ds(j * 16, 16)
                    hbuf[b, r, sl] = jnp.maximum(
                        hbuf[b, r, sl] + ebuf[b, r, sl], 0.0)

        def issue_s(j, b):
            hj = lax.rem(lax.div(j, IB), 2)
            rj = lax.rem(j, IB)
            pltpu.async_copy(hbuf.at[b], acc.at[dblk.at[hj, rj]], ssems[b],
                             add=True)

        def wait_s(b):
            pltpu.make_async_copy(hbuf.at[b], acc.at[dblk.at[0, 0]],
                                  ssems[b]).wait()

        load_block(0)
        issue_g(0, 0)
        issue_g(1, 1)

        @pl.loop(0, NCHUNK // 3)
        def _(p):
            for b in range(3):
                i = 3 * p + b
                nb = (b + 2) % 3
                wait_g(b)
                compute(b)
                issue_s(i, b)

                @pl.when(i >= 1)
                def _():
                    wait_s(nb)

                nxt = i + 2

                @pl.when(jnp.logical_and(lax.rem(nxt, IB) == 0,
                                         nxt < NCHUNK))
                def _():
                    wait_block()

                @pl.when(jnp.logical_and(lax.rem(i, IB) == 0,
                                         i + IB < NCHUNK))
                def _():
                    prefetch_block(lax.div(i, IB) + 1)

                @pl.when(nxt < NCHUNK)
                def _():
                    issue_g(nxt, nb)

        for i_e in range(3 * (NCHUNK // 3), NCHUNK):
            b = i_e % 3
            wait_g(b)
            compute(b)
            issue_s(i_e, b)
            wait_s((i_e - 1) % 3)
        wait_s((NCHUNK - 1) % 3)
        plsc.subcore_barrier()

        @pl.when(s < NS - 1)
        def _():
            for kblk in range(RPS // DR):
                r0 = s * RPS + kblk * DR
                pltpu.sync_copy(acc.at[pl.ds(r0, DR)],
                                out_hbm.at[pl.ds(c * N + r0, DR)])

        @pl.when(s == NS - 1)
        def _():
            base = (NS - 1) * RPS
            for r0, sz in ((0, 128), (128, 128), (256, 128), (384, 16)):
                pltpu.sync_copy(acc.at[pl.ds(base + r0, sz)],
                                out_hbm.at[pl.ds(c * N + base + r0, sz)])

    return k(hlo, hhi, src4, dst4, typ4, elo, ehi)


def _tc_input(node, W_in, b_in):
    B = 1000

    def body(x_ref, w_ref, b_ref, out_ref):
        z = jnp.dot(x_ref[...], w_ref[...],
                    preferred_element_type=jnp.float32) + b_ref[...]
        out_ref[0] = z[:, :HH]
        out_ref[1] = z[:, HH:]

    return pl.pallas_call(
        body,
        grid=(N // B,),
        in_specs=[
            pl.BlockSpec((B, D_IN), lambda j: (j, 0)),
            pl.BlockSpec((D_IN, H), lambda j: (0, 0)),
            pl.BlockSpec((1, H), lambda j: (0, 0)),
        ],
        out_specs=pl.BlockSpec((2, B, HH), lambda j: (0, j, 0)),
        out_shape=jax.ShapeDtypeStruct((2, N, HH), jnp.float32),
    )(node, W_in, b_in.reshape(1, H))


def _tc_layer(h2, agg2, Wi2, bi):
    B = 1000

    def body(h_ref, a_ref, w_ref, b_ref, out_ref):
        h_lo = h_ref[0]
        h_hi = h_ref[1]
        x_lo = h_lo + a_ref[0]
        x_hi = h_hi + a_ref[1]
        z = (jnp.dot(x_lo, w_ref[0], preferred_element_type=jnp.float32)
             + jnp.dot(x_hi, w_ref[1], preferred_element_type=jnp.float32)
             + b_ref[...])
        z = jnp.where(z > 0, z, 0.01 * z)
        out_ref[0] = z[:, :HH] + h_lo
        out_ref[1] = z[:, HH:] + h_hi

    return pl.pallas_call(
        body,
        grid=(N // B,),
        in_specs=[
            pl.BlockSpec((2, B, HH), lambda j: (0, j, 0)),
            pl.BlockSpec((2, B, HH), lambda j: (0, j, 0)),
            pl.BlockSpec((2, HH, H), lambda j: (0, 0, 0)),
            pl.BlockSpec((1, H), lambda j: (0, 0)),
        ],
        out_specs=pl.BlockSpec((2, B, HH), lambda j: (0, j, 0)),
        out_shape=jax.ShapeDtypeStruct((2, N, HH), jnp.float32),
    )(h2, agg2, Wi2, bi)


def _tc_pool(h2, bi_row, W_out, b_out):
    B = 1000

    def body(h_ref, bi_ref, w_ref, b_ref, out_ref, sums, counts):
        j = pl.program_id(0)

        @pl.when(j == 0)
        def _():
            sums[...] = jnp.zeros_like(sums)
            counts[...] = jnp.zeros_like(counts)

        x = jnp.concatenate([h_ref[0], h_ref[1]], axis=1)
        gids = lax.broadcasted_iota(jnp.int32, (G, B), 0)
        m = (gids == bi_ref[0]).astype(jnp.float32)
        sums[...] += jnp.dot(m, x, preferred_element_type=jnp.float32)
        counts[...] += jnp.sum(m, axis=1, keepdims=True)

        @pl.when(j == pl.num_programs(0) - 1)
        def _():
            mean = sums[...] / jnp.maximum(counts[...], 1.0)
            out_ref[...] = jnp.dot(mean, w_ref[...],
                                   preferred_element_type=jnp.float32) + b_ref[...]

    return pl.pallas_call(
        body,
        grid=(N // B,),
        in_specs=[
            pl.BlockSpec((2, B, HH), lambda j: (0, j, 0)),
            pl.BlockSpec((1, 1, B), lambda j: (j, 0, 0)),
            pl.BlockSpec((H, 1), lambda j: (0, 0)),
            pl.BlockSpec((1, 1), lambda j: (0, 0)),
        ],
        out_specs=pl.BlockSpec((G, 1), lambda j: (0, 0)),
        out_shape=jax.ShapeDtypeStruct((G, 1), jnp.float32),
        scratch_shapes=[
            pltpu.VMEM((G, H), jnp.float32),
            pltpu.VMEM((G, 1), jnp.float32),
        ],
    )(h2, bi_row.reshape(N // B, 1, B), W_out, b_out.reshape(1, 1))


def kernel(node, edge, edge_type, batch_index, W_in, b_in, emb, Wc, bc, W_out, b_out):
    src4 = edge[:, 0].reshape(NS, NBLK, IB, CH)
    dst4 = edge[:, 1].reshape(NS, NBLK, IB, CH)
    typ4 = edge_type[:, 0].reshape(NS, NBLK, IB, CH)
    elo = emb[:, :HH]
    ehi = emb[:, HH:]

    h2 = _tc_input(node, W_in, b_in)
    for i in range(L):
        agg = _sc_layer(h2[0], h2[1], src4, dst4, typ4, elo, ehi)
        h2 = _tc_layer(h2, agg.reshape(2, N, HH),
                       Wc[i].reshape(2, HH, H), bc[i].reshape(1, H))
    out = _tc_pool(h2, batch_index.reshape(1, N), W_out, b_out.reshape(1, 1))
    return out.reshape(G)


# speedup vs baseline: 1.0842x; 1.0683x over previous
import functools

import jax
import jax.numpy as jnp
from jax import lax
from jax.experimental import pallas as pl
from jax.experimental.pallas import tpu as pltpu
from jax.experimental.pallas import tpu_sc as plsc

N = 10000
E = 320000
D_IN = 128
H = 256
HH = 128
L = 4
G = 64
T = 100

NS = 16
EPT = E // NS
CH = 40
NCHUNK = EPT // CH
IB = 10
NBLK = NCHUNK // IB
NPAD = 10240
RPS = NPAD // NS
ZR = 16
DR = 128


def _sc_layer(hlo, hhi, src4, dst4, typ4, elo, ehi):
    mesh = plsc.VectorSubcoreMesh(core_axis_name="c", subcore_axis_name="s")

    @functools.partial(
        pl.kernel,
        out_type=jax.ShapeDtypeStruct((2 * N, HH), jnp.float32),
        mesh=mesh,
        scratch_types=[
            pltpu.VMEM((2, IB, CH), jnp.int32),
            pltpu.VMEM((2, IB, CH), jnp.int32),
            pltpu.VMEM((2, IB, CH), jnp.int32),
            pltpu.VMEM((3, CH, HH), jnp.float32),
            pltpu.VMEM((3, CH, HH), jnp.float32),
            pltpu.VMEM_SHARED((NPAD, HH), jnp.float32),
            pltpu.VMEM_SHARED((104, HH), jnp.float32),
            pltpu.SemaphoreType.DMA, pltpu.SemaphoreType.DMA,
            pltpu.SemaphoreType.DMA, pltpu.SemaphoreType.DMA,
            pltpu.SemaphoreType.DMA, pltpu.SemaphoreType.DMA,
            pltpu.SemaphoreType.DMA, pltpu.SemaphoreType.DMA,
            pltpu.SemaphoreType.DMA, pltpu.SemaphoreType.DMA,
        ],
    )
    def k(hlo_hbm, hhi_hbm, src_hbm, dst_hbm, typ_hbm, elo_hbm, ehi_hbm,
          out_hbm, sblk, dblk, tblk, hbuf, ebuf, acc, esp,
          gs0, gs1, gs2, es0, es1, es2, ss0, ss1, ss2, isem):
        gsems = (gs0, gs1, gs2)
        esems = (es0, es1, es2)
        ssems = (ss0, ss1, ss2)
        c = lax.axis_index("c")
        s = lax.axis_index("s")

        @pl.when(s == 0)
        def _():
            @pl.when(c == 0)
            def _():
                pltpu.sync_copy(elo_hbm, esp.at[pl.ds(0, T)])

            @pl.when(c == 1)
            def _():
                pltpu.sync_copy(ehi_hbm, esp.at[pl.ds(0, T)])

        zero16 = jnp.zeros((16,), jnp.float32)

        @pl.loop(0, CH)
        def _(r):
            for j in range(HH // 16):
                ebuf[0, r, pl.ds(j * 16, 16)] = zero16

        for kb in range(RPS // CH):
            pltpu.sync_copy(ebuf.at[0], acc.at[pl.ds(s * RPS + kb * CH, CH)])
        plsc.subcore_barrier()

        def load_block(kb):
            hh = lax.rem(kb, 2)
            pltpu.sync_copy(src_hbm.at[s, kb], sblk.at[hh])
            pltpu.sync_copy(dst_hbm.at[s, kb], dblk.at[hh])
            pltpu.sync_copy(typ_hbm.at[s, kb], tblk.at[hh])

        def prefetch_block(kb):
            hh = lax.rem(kb, 2)
            pltpu.async_copy(src_hbm.at[s, kb], sblk.at[hh], isem)
            pltpu.async_copy(dst_hbm.at[s, kb], dblk.at[hh], isem)
            pltpu.async_copy(typ_hbm.at[s, kb], tblk.at[hh], isem)

        def wait_block():
            pltpu.make_async_copy(src_hbm.at[s, 0], sblk.at[0], isem).wait()
            pltpu.make_async_copy(dst_hbm.at[s, 0], dblk.at[0], isem).wait()
            pltpu.make_async_copy(typ_hbm.at[s, 0], tblk.at[0], isem).wait()

        def issue_g(j, b):
            hj = lax.rem(lax.div(j, IB), 2)
            rj = lax.rem(j, IB)
            pltpu.async_copy(esp.at[tblk.at[hj, rj]], ebuf.at[b], esems[b])

            @pl.when(c == 0)
            def _():
                pltpu.async_copy(hlo_hbm.at[sblk.at[hj, rj]], hbuf.at[b],
                                 gsems[b])

            @pl.when(c == 1)
            def _():
                pltpu.async_copy(hhi_hbm.at[sblk.at[hj, rj]], hbuf.at[b],
                                 gsems[b])

        def wait_g(b):
            pltpu.make_async_copy(hlo_hbm.at[sblk.at[0, 0]], hbuf.at[b],
                                  gsems[b]).wait()
            pltpu.make_async_copy(esp.at[tblk.at[0, 0]], ebuf.at[b],
                                  esems[b]).wait()

        def compute(b):
            @plsc.parallel_loop(0, CH, unroll=2)
            def _(r):
                for j in range(HH // 16):
                    sl = pl.ds(j * 16, 16)
                    hbuf[b, r, sl] = jnp.maximum(
                        hbuf[b, r, sl] + ebuf[b, r, sl], 0.0)

        def issue_s(j, b):
            hj = lax.rem(lax.div(j, IB), 2)
            rj = lax.rem(j, IB)
            pltpu.async_copy(hbuf.at[b], acc.at[dblk.at[hj, rj]], ssems[b],
                             add=True)

        def wait_s(b):
            pltpu.make_async_copy(hbuf.at[b], acc.at[dblk.at[0, 0]],
                                  ssems[b]).wait()

        load_block(0)
        issue_g(0, 0)
        issue_g(1, 1)

        @pl.loop(0, NCHUNK // 3)
        def _(p):
            for b in range(3):
                i = 3 * p + b
                nb = (b + 2) % 3
                wait_g(b)
                compute(b)
                issue_s(i, b)

                @pl.when(i >= 1)
                def _():
                    wait_s(nb)

                nxt = i + 2

                @pl.when(jnp.logical_and(lax.rem(nxt, IB) == 0,
                                         nxt < NCHUNK))
                def _():
                    wait_block()

                @pl.when(jnp.logical_and(lax.rem(i, IB) == 0,
                                         i + IB < NCHUNK))
                def _():
                    prefetch_block(lax.div(i, IB) + 1)

                @pl.when(nxt < NCHUNK)
                def _():
                    issue_g(nxt, nb)

        for i_e in range(3 * (NCHUNK // 3), NCHUNK):
            b = i_e % 3
            wait_g(b)
            compute(b)
            issue_s(i_e, b)
            wait_s((i_e - 1) % 3)
        wait_s((NCHUNK - 1) % 3)
        plsc.subcore_barrier()

        @pl.when(s < NS - 1)
        def _():
            for kblk in range(RPS // DR):
                r0 = s * RPS + kblk * DR
                pltpu.sync_copy(acc.at[pl.ds(r0, DR)],
                                out_hbm.at[pl.ds(c * N + r0, DR)])

        @pl.when(s == NS - 1)
        def _():
            base = (NS - 1) * RPS
            for r0, sz in ((0, 128), (128, 128), (256, 128), (384, 16)):
                pltpu.sync_copy(acc.at[pl.ds(base + r0, sz)],
                                out_hbm.at[pl.ds(c * N + base + r0, sz)])

    return k(hlo, hhi, src4, dst4, typ4, elo, ehi)


def _tc_input(node, W_in, b_in):
    B = 1000

    def body(x_ref, w_ref, b_ref, out_ref):
        z = jnp.dot(x_ref[...], w_ref[...],
                    preferred_element_type=jnp.float32) + b_ref[...]
        out_ref[0] = z[:, :HH]
        out_ref[1] = z[:, HH:]

    return pl.pallas_call(
        body,
        grid=(N // B,),
        in_specs=[
            pl.BlockSpec((B, D_IN), lambda j: (j, 0)),
            pl.BlockSpec((D_IN, H), lambda j: (0, 0)),
            pl.BlockSpec((1, H), lambda j: (0, 0)),
        ],
        out_specs=pl.BlockSpec((2, B, HH), lambda j: (0, j, 0)),
        out_shape=jax.ShapeDtypeStruct((2, N, HH), jnp.float32),
    )(node, W_in, b_in.reshape(1, H))


def _tc_layer(h2, agg2, Wi2, bi):
    B = 1000

    def body(h_ref, a_ref, w_ref, b_ref, out_ref):
        h_lo = h_ref[0]
        h_hi = h_ref[1]
        x_lo = h_lo + a_ref[0]
        x_hi = h_hi + a_ref[1]
        z = (jnp.dot(x_lo, w_ref[0], preferred_element_type=jnp.float32)
             + jnp.dot(x_hi, w_ref[1], preferred_element_type=jnp.float32)
             + b_ref[...])
        z = jnp.where(z > 0, z, 0.01 * z)
        out_ref[0] = z[:, :HH] + h_lo
        out_ref[1] = z[:, HH:] + h_hi

    return pl.pallas_call(
        body,
        grid=(N // B,),
        in_specs=[
            pl.BlockSpec((2, B, HH), lambda j: (0, j, 0)),
            pl.BlockSpec((2, B, HH), lambda j: (0, j, 0)),
            pl.BlockSpec((2, HH, H), lambda j: (0, 0, 0)),
            pl.BlockSpec((1, H), lambda j: (0, 0)),
        ],
        out_specs=pl.BlockSpec((2, B, HH), lambda j: (0, j, 0)),
        out_shape=jax.ShapeDtypeStruct((2, N, HH), jnp.float32),
    )(h2, agg2, Wi2, bi)


def _tc_pool(h2, bi_row, W_out, b_out):
    B = 1000

    def body(h_ref, bi_ref, w_ref, b_ref, out_ref, sums, counts):
        j = pl.program_id(0)

        @pl.when(j == 0)
        def _():
            sums[...] = jnp.zeros_like(sums)
            counts[...] = jnp.zeros_like(counts)

        x = jnp.concatenate([h_ref[0], h_ref[1]], axis=1)
        gids = lax.broadcasted_iota(jnp.int32, (G, B), 0)
        m = (gids == bi_ref[0]).astype(jnp.float32)
        sums[...] += jnp.dot(m, x, preferred_element_type=jnp.float32)
        counts[...] += jnp.sum(m, axis=1, keepdims=True)

        @pl.when(j == pl.num_programs(0) - 1)
        def _():
            mean = sums[...] / jnp.maximum(counts[...], 1.0)
            out_ref[...] = jnp.dot(mean, w_ref[...],
                                   preferred_element_type=jnp.float32) + b_ref[...]

    return pl.pallas_call(
        body,
        grid=(N // B,),
        in_specs=[
            pl.BlockSpec((2, B, HH), lambda j: (0, j, 0)),
            pl.BlockSpec((1, 1, B), lambda j: (j, 0, 0)),
            pl.BlockSpec((H, 1), lambda j: (0, 0)),
            pl.BlockSpec((1, 1), lambda j: (0, 0)),
        ],
        out_specs=pl.BlockSpec((G, 1), lambda j: (0, 0)),
        out_shape=jax.ShapeDtypeStruct((G, 1), jnp.float32),
        scratch_shapes=[
            pltpu.VMEM((G, H), jnp.float32),
            pltpu.VMEM((G, 1), jnp.float32),
        ],
    )(h2, bi_row.reshape(N // B, 1, B), W_out, b_out.reshape(1, 1))


def kernel(node, edge, edge_type, batch_index, W_in, b_in, emb, Wc, bc, W_out, b_out):
    src4 = edge[:, 0].reshape(NS, NBLK, IB, CH)
    dst4 = edge[:, 1].reshape(NS, NBLK, IB, CH)
    typ4 = edge_type[:, 0].reshape(NS, NBLK, IB, CH)
    elo = emb[:, :HH]
    ehi = emb[:, HH:]

    h2 = _tc_input(node, W_in, b_in)
    for i in range(L):
        agg = _sc_layer(h2[0], h2[1], src4, dst4, typ4, elo, ehi)
        h2 = _tc_layer(h2, agg.reshape(2, N, HH),
                       Wc[i].reshape(2, HH, H), bc[i].reshape(1, H))
    out = _tc_pool(h2, batch_index.reshape(1, N), W_out, b_out.reshape(1, 1))
    return out.reshape(G)


# speedup vs baseline: 1.1285x; 1.0409x over previous
import functools

import jax
import jax.numpy as jnp
from jax import lax
from jax.experimental import pallas as pl
from jax.experimental.pallas import tpu as pltpu
from jax.experimental.pallas import tpu_sc as plsc

N = 10000
E = 320000
D_IN = 128
H = 256
HH = 128
L = 4
G = 64
T = 100

NS = 16
EPT = E // NS
CH = 50
NCHUNK = EPT // CH
IB = 8
NBLK = NCHUNK // IB
NPAD = 10112
RPS = NPAD // NS


def _sc_layer(hlo, hhi, src4, dst4, typ4, elo, ehi):
    mesh = plsc.VectorSubcoreMesh(core_axis_name="c", subcore_axis_name="s")

    @functools.partial(
        pl.kernel,
        out_type=jax.ShapeDtypeStruct((2 * N, HH), jnp.float32),
        mesh=mesh,
        scratch_types=[
            pltpu.VMEM((2, IB, CH), jnp.int32),
            pltpu.VMEM((2, IB, CH), jnp.int32),
            pltpu.VMEM((2, IB, CH), jnp.int32),
            pltpu.VMEM((3, CH, HH), jnp.float32),
            pltpu.VMEM((3, CH, HH), jnp.float32),
            pltpu.VMEM_SHARED((NPAD, HH), jnp.float32),
            pltpu.VMEM_SHARED((104, HH), jnp.float32),
            pltpu.SemaphoreType.DMA, pltpu.SemaphoreType.DMA,
            pltpu.SemaphoreType.DMA, pltpu.SemaphoreType.DMA,
            pltpu.SemaphoreType.DMA, pltpu.SemaphoreType.DMA,
            pltpu.SemaphoreType.DMA, pltpu.SemaphoreType.DMA,
            pltpu.SemaphoreType.DMA, pltpu.SemaphoreType.DMA,
        ],
    )
    def k(hlo_hbm, hhi_hbm, src_hbm, dst_hbm, typ_hbm, elo_hbm, ehi_hbm,
          out_hbm, sblk, dblk, tblk, hbuf, ebuf, acc, esp,
          gs0, gs1, gs2, es0, es1, es2, ss0, ss1, ss2, isem):
        gsems = (gs0, gs1, gs2)
        esems = (es0, es1, es2)
        ssems = (ss0, ss1, ss2)
        c = lax.axis_index("c")
        s = lax.axis_index("s")

        @pl.when(s == 0)
        def _():
            @pl.when(c == 0)
            def _():
                pltpu.sync_copy(elo_hbm, esp.at[pl.ds(0, T)])

            @pl.when(c == 1)
            def _():
                pltpu.sync_copy(ehi_hbm, esp.at[pl.ds(0, T)])

        zero16 = jnp.zeros((16,), jnp.float32)

        @pl.loop(0, CH)
        def _(r):
            for j in range(HH // 16):
                ebuf[0, r, pl.ds(j * 16, 16)] = zero16

        for kz in range(13):
            pltpu.sync_copy(ebuf.at[0, pl.ds(0, 48)],
                            acc.at[pl.ds(s * RPS + kz * 48, 48)])
        pltpu.sync_copy(ebuf.at[0, pl.ds(0, 8)],
                        acc.at[pl.ds(s * RPS + 624, 8)])
        plsc.subcore_barrier()

        def load_block(kb):
            hh = lax.rem(kb, 2)
            pltpu.sync_copy(src_hbm.at[s, kb], sblk.at[hh])
            pltpu.sync_copy(dst_hbm.at[s, kb], dblk.at[hh])
            pltpu.sync_copy(typ_hbm.at[s, kb], tblk.at[hh])

        def prefetch_block(kb):
            hh = lax.rem(kb, 2)
            pltpu.async_copy(src_hbm.at[s, kb], sblk.at[hh], isem)
            pltpu.async_copy(dst_hbm.at[s, kb], dblk.at[hh], isem)
            pltpu.async_copy(typ_hbm.at[s, kb], tblk.at[hh], isem)

        def wait_block():
            pltpu.make_async_copy(src_hbm.at[s, 0], sblk.at[0], isem).wait()
            pltpu.make_async_copy(dst_hbm.at[s, 0], dblk.at[0], isem).wait()
            pltpu.make_async_copy(typ_hbm.at[s, 0], tblk.at[0], isem).wait()

        def issue_g(j, b):
            hj = lax.rem(lax.div(j, IB), 2)
            rj = lax.rem(j, IB)
            pltpu.async_copy(esp.at[tblk.at[hj, rj]], ebuf.at[b], esems[b])

            @pl.when(c == 0)
            def _():
                pltpu.async_copy(hlo_hbm.at[sblk.at[hj, rj]], hbuf.at[b],
                                 gsems[b])

            @pl.when(c == 1)
            def _():
                pltpu.async_copy(hhi_hbm.at[sblk.at[hj, rj]], hbuf.at[b],
                                 gsems[b])

        def wait_g(b):
            pltpu.make_async_copy(hlo_hbm.at[sblk.at[0, 0]], hbuf.at[b],
                                  gsems[b]).wait()
            pltpu.make_async_copy(esp.at[tblk.at[0, 0]], ebuf.at[b],
                                  esems[b]).wait()

        def compute(b):
            @plsc.parallel_loop(0, CH, unroll=5)
            def _(r):
                for j in range(HH // 16):
                    sl = pl.ds(j * 16, 16)
                    hbuf[b, r, sl] = jnp.maximum(
                        hbuf[b, r, sl] + ebuf[b, r, sl], 0.0)

        def issue_s(j, b):
            hj = lax.rem(lax.div(j, IB), 2)
            rj = lax.rem(j, IB)
            pltpu.async_copy(hbuf.at[b], acc.at[dblk.at[hj, rj]], ssems[b],
                             add=True)

        def wait_s(b):
            pltpu.make_async_copy(hbuf.at[b], acc.at[dblk.at[0, 0]],
                                  ssems[b]).wait()

        load_block(0)
        issue_g(0, 0)
        issue_g(1, 1)

        @pl.loop(0, NCHUNK // 3)
        def _(p):
            for b in range(3):
                i = 3 * p + b
                nb = (b + 2) % 3
                wait_g(b)
                compute(b)
                issue_s(i, b)

                @pl.when(i >= 1)
                def _():
                    wait_s(nb)

                nxt = i + 2

                @pl.when(jnp.logical_and(lax.rem(nxt, IB) == 0,
                                         nxt < NCHUNK))
                def _():
                    wait_block()

                @pl.when(jnp.logical_and(lax.rem(i, IB) == 0,
                                         i + IB < NCHUNK))
                def _():
                    prefetch_block(lax.div(i, IB) + 1)

                @pl.when(nxt < NCHUNK)
                def _():
                    issue_g(nxt, nb)

        for i_e in range(3 * (NCHUNK // 3), NCHUNK):
            b = i_e % 3
            wait_g(b)
            compute(b)
            issue_s(i_e, b)
            wait_s((i_e - 1) % 3)
        wait_s((NCHUNK - 1) % 3)
        plsc.subcore_barrier()

        @pl.when(s < NS - 1)
        def _():
            for r0, sz in ((0, 128), (128, 128), (256, 128), (384, 128),
                           (512, 120)):
                q0 = s * RPS + r0
                pltpu.sync_copy(acc.at[pl.ds(q0, sz)],
                                out_hbm.at[pl.ds(c * N + q0, sz)])

        @pl.when(s == NS - 1)
        def _():
            base = (NS - 1) * RPS
            for r0, sz in ((0, 128), (128, 128), (256, 128), (384, 128),
                           (512, 8)):
                pltpu.sync_copy(acc.at[pl.ds(base + r0, sz)],
                                out_hbm.at[pl.ds(c * N + base + r0, sz)])

    return k(hlo, hhi, src4, dst4, typ4, elo, ehi)


def _tc_input(node, W_in, b_in):
    B = 1000

    def body(x_ref, w_ref, b_ref, out_ref):
        z = jnp.dot(x_ref[...], w_ref[...],
                    preferred_element_type=jnp.float32) + b_ref[...]
        out_ref[0] = z[:, :HH]
        out_ref[1] = z[:, HH:]

    return pl.pallas_call(
        body,
        grid=(N // B,),
        in_specs=[
            pl.BlockSpec((B, D_IN), lambda j: (j, 0)),
            pl.BlockSpec((D_IN, H), lambda j: (0, 0)),
            pl.BlockSpec((1, H), lambda j: (0, 0)),
        ],
        out_specs=pl.BlockSpec((2, B, HH), lambda j: (0, j, 0)),
        out_shape=jax.ShapeDtypeStruct((2, N, HH), jnp.float32),
    )(node, W_in, b_in.reshape(1, H))


def _tc_layer(h2, agg2, Wi2, bi):
    B = 1000

    def body(h_ref, a_ref, w_ref, b_ref, out_ref):
        h_lo = h_ref[0]
        h_hi = h_ref[1]
        x_lo = h_lo + a_ref[0]
        x_hi = h_hi + a_ref[1]
        z = (jnp.dot(x_lo, w_ref[0], preferred_element_type=jnp.float32)
             + jnp.dot(x_hi, w_ref[1], preferred_element_type=jnp.float32)
             + b_ref[...])
        z = jnp.where(z > 0, z, 0.01 * z)
        out_ref[0] = z[:, :HH] + h_lo
        out_ref[1] = z[:, HH:] + h_hi

    return pl.pallas_call(
        body,
        grid=(N // B,),
        in_specs=[
            pl.BlockSpec((2, B, HH), lambda j: (0, j, 0)),
            pl.BlockSpec((2, B, HH), lambda j: (0, j, 0)),
            pl.BlockSpec((2, HH, H), lambda j: (0, 0, 0)),
            pl.BlockSpec((1, H), lambda j: (0, 0)),
        ],
        out_specs=pl.BlockSpec((2, B, HH), lambda j: (0, j, 0)),
        out_shape=jax.ShapeDtypeStruct((2, N, HH), jnp.float32),
    )(h2, agg2, Wi2, bi)


def _tc_pool(h2, bi_row, W_out, b_out):
    B = 1000

    def body(h_ref, bi_ref, w_ref, b_ref, out_ref, sums, counts):
        j = pl.program_id(0)

        @pl.when(j == 0)
        def _():
            sums[...] = jnp.zeros_like(sums)
            counts[...] = jnp.zeros_like(counts)

        x = jnp.concatenate([h_ref[0], h_ref[1]], axis=1)
        gids = lax.broadcasted_iota(jnp.int32, (G, B), 0)
        m = (gids == bi_ref[0]).astype(jnp.float32)
        sums[...] += jnp.dot(m, x, preferred_element_type=jnp.float32)
        counts[...] += jnp.sum(m, axis=1, keepdims=True)

        @pl.when(j == pl.num_programs(0) - 1)
        def _():
            mean = sums[...] / jnp.maximum(counts[...], 1.0)
            out_ref[...] = jnp.dot(mean, w_ref[...],
                                   preferred_element_type=jnp.float32) + b_ref[...]

    return pl.pallas_call(
        body,
        grid=(N // B,),
        in_specs=[
            pl.BlockSpec((2, B, HH), lambda j: (0, j, 0)),
            pl.BlockSpec((1, 1, B), lambda j: (j, 0, 0)),
            pl.BlockSpec((H, 1), lambda j: (0, 0)),
            pl.BlockSpec((1, 1), lambda j: (0, 0)),
        ],
        out_specs=pl.BlockSpec((G, 1), lambda j: (0, 0)),
        out_shape=jax.ShapeDtypeStruct((G, 1), jnp.float32),
        scratch_shapes=[
            pltpu.VMEM((G, H), jnp.float32),
            pltpu.VMEM((G, 1), jnp.float32),
        ],
    )(h2, bi_row.reshape(N // B, 1, B), W_out, b_out.reshape(1, 1))


def kernel(node, edge, edge_type, batch_index, W_in, b_in, emb, Wc, bc, W_out, b_out):
    src4 = edge[:, 0].reshape(NS, NBLK, IB, CH)
    dst4 = edge[:, 1].reshape(NS, NBLK, IB, CH)
    typ4 = edge_type[:, 0].reshape(NS, NBLK, IB, CH)
    elo = emb[:, :HH]
    ehi = emb[:, HH:]

    h2 = _tc_input(node, W_in, b_in)
    for i in range(L):
        agg = _sc_layer(h2[0], h2[1], src4, dst4, typ4, elo, ehi)
        h2 = _tc_layer(h2, agg.reshape(2, N, HH),
                       Wc[i].reshape(2, HH, H), bc[i].reshape(1, H))
    out = _tc_pool(h2, batch_index.reshape(1, N), W_out, b_out.reshape(1, 1))
    return out.reshape(G)
